# baseline (speedup 1.0000x reference)
import functools

import jax
import jax.numpy as jnp
from jax import lax
from jax.experimental import pallas as pl
from jax.experimental.pallas import tpu as pltpu
from jax.experimental.pallas import tpu_sc as plsc

N = 10000
E = 320000
D = 128
N_COMM = 64

NC = 2
NS = 16
NW = NC * NS
L = 16
EPT = E // NW
CHUNK = 128
NCHUNK = 79
EPT_PAD = NCHUNK * CHUNK
DPAD = 10120
N_PAD = 10240
PAD0 = 120
HALF = N_PAD // 2
HRPT = HALF // NS
RPT = N_PAD // NS
ZCH = 160

_mesh = plsc.VectorSubcoreMesh(core_axis_name="c", subcore_axis_name="s")


@functools.partial(
    pl.kernel,
    mesh=_mesh,
    out_type=jax.ShapeDtypeStruct((NW, N), jnp.float32),
    compiler_params=pltpu.CompilerParams(needs_layout_passes=False),
    scratch_types=[
        pltpu.VMEM((EPT,), jnp.int32),
        pltpu.VMEM((N,), jnp.float32),
    ],
)
def _sc_degree(dst_hbm, out_hbm, idx_v, hist):
    c = lax.axis_index("c")
    s = lax.axis_index("s")
    wid = c * NS + s

    def _fill(i, _):
        hist[pl.ds(i * L, L)] = jnp.zeros((L,), jnp.float32)
        return 0

    lax.fori_loop(0, N // L, _fill, 0)
    pltpu.sync_copy(dst_hbm.at[pl.ds(wid * EPT, EPT)], idx_v)
    ones = jnp.ones((L,), jnp.float32)

    def _step(j, _):
        d = idx_v[pl.ds(j * L, L)]
        plsc.addupdate_scatter(hist, [d], ones)
        return 0

    lax.fori_loop(0, EPT // L, _step, 0)
    pltpu.sync_copy(hist, out_hbm.at[wid])


@functools.partial(
    pl.kernel,
    mesh=_mesh,
    out_type=jax.ShapeDtypeStruct((NC, N_PAD, D), jnp.float32),
    scratch_types=[
        pltpu.VMEM((NCHUNK, CHUNK), jnp.int32),
        pltpu.VMEM((NCHUNK, CHUNK), jnp.int32),
        pltpu.VMEM((NCHUNK, CHUNK), jnp.int32),
        pltpu.VMEM((CHUNK, D), jnp.float32),
        pltpu.VMEM((CHUNK, D), jnp.float32),
        pltpu.VMEM((ZCH, D), jnp.float32),
        pltpu.VMEM_SHARED((HALF, D), jnp.float32),
        pltpu.SemaphoreType.DMA,
        pltpu.SemaphoreType.DMA,
        pltpu.SemaphoreType.DMA,
        pltpu.SemaphoreType.DMA,
    ],
)
def _sc_scatter(g_hbm, src_hbm, dst_hbm, out_hbm,
                src_v, dst_v, loc_v, rows_a, rows_b, zero_v, acc, sem_a,
                sem_b, sad_a, sad_b):
    c = lax.axis_index("c")
    s = lax.axis_index("s")
    wid = c * NS + s

    def _fill(i, _):
        for k in range(D // L):
            zero_v[i, pl.ds(k * L, L)] = jnp.zeros((L,), jnp.float32)
        return 0

    lax.fori_loop(0, ZCH, _fill, 0)
    pltpu.sync_copy(src_hbm.at[wid], src_v)
    pltpu.sync_copy(dst_hbm.at[wid], dst_v)

    for p in range(2):
        base = p * HALF

        trash = (HALF - 1) * p

        def _localize(j, _):
            for q in range(CHUNK // L):
                r = dst_v[j, pl.ds(q * L, L)] + (PAD0 - base)
                ok = (r >= 0) & (r < HALF)
                loc_v[j, pl.ds(q * L, L)] = jnp.where(ok, r, trash)
            return 0

        lax.fori_loop(0, NCHUNK, _localize, 0)

        for t in range(HRPT // ZCH):
            pltpu.sync_copy(zero_v, acc.at[pl.ds(s * HRPT + t * ZCH, ZCH)])
        plsc.subcore_barrier()

        def wait_g(rows, sem):
            pltpu.make_async_copy(g_hbm.at[src_v.at[0]], rows, sem).wait()

        def wait_a(rows, sem):
            pltpu.make_async_copy(rows, acc.at[loc_v.at[0]], sem).wait()

        pltpu.async_copy(g_hbm.at[src_v.at[0]], rows_a, sem_a)
        wait_g(rows_a, sem_a)
        pltpu.async_copy(rows_a, acc.at[loc_v.at[0]], sad_a, add=True)
        pltpu.async_copy(g_hbm.at[src_v.at[1]], rows_b, sem_b)
        wait_g(rows_b, sem_b)
        pltpu.async_copy(rows_b, acc.at[loc_v.at[1]], sad_b, add=True)
        wait_a(rows_a, sad_a)
        pltpu.async_copy(g_hbm.at[src_v.at[2]], rows_a, sem_a)

        def _step(i, _):
            j = 2 * i
            wait_g(rows_a, sem_a)
            pltpu.async_copy(rows_a, acc.at[loc_v.at[j]], sad_a, add=True)
            wait_a(rows_b, sad_b)
            pltpu.async_copy(g_hbm.at[src_v.at[j + 1]], rows_b, sem_b)
            wait_g(rows_b, sem_b)
            pltpu.async_copy(rows_b, acc.at[loc_v.at[j + 1]], sad_b, add=True)
            wait_a(rows_a, sad_a)
            pltpu.async_copy(g_hbm.at[src_v.at[j + 2]], rows_a, sem_a)
            return 0

        lax.fori_loop(1, (NCHUNK - 1) // 2, _step, 0)
        wait_g(rows_a, sem_a)
        pltpu.async_copy(rows_a, acc.at[loc_v.at[NCHUNK - 1]], sad_a, add=True)
        wait_a(rows_b, sad_b)
        wait_a(rows_a, sad_a)
        plsc.subcore_barrier()
        pltpu.sync_copy(acc.at[pl.ds(s * HRPT, HRPT)],
                        out_hbm.at[c, pl.ds(base + s * HRPT, HRPT)])
        plsc.subcore_barrier()


ROWS = 1000


def _dinv(deg_ref):
    deg = jnp.sum(deg_ref[...], axis=1) + 1.0
    return lax.rsqrt(deg)


def _tc_in(x_ref, w_ref, deg_ref, o_ref):
    di = _dinv(deg_ref)
    o_ref[...] = jnp.dot(x_ref[...], w_ref[...],
                         preferred_element_type=jnp.float32) * di[:, None]


def _tc_mid(s0_ref, s1_ref, g_ref, deg_ref, w_ref, b_ref, o_ref):
    di = _dinv(deg_ref)
    h = (s0_ref[0] + s1_ref[0] + g_ref[...]) * di[:, None] + b_ref[...]
    h = jnp.maximum(h, 0.0)
    o_ref[...] = jnp.dot(h, w_ref[...],
                         preferred_element_type=jnp.float32) * di[:, None]


def _tc_out(s0_ref, s1_ref, g_ref, deg_ref, w_ref, b_ref, bfc_ref, o_ref):
    di = _dinv(deg_ref)
    h = (s0_ref[0] + s1_ref[0] + g_ref[...]) * di[:, None] + b_ref[...]
    h = jnp.maximum(h, 0.0)
    z = jnp.dot(h, w_ref[...], preferred_element_type=jnp.float32) + bfc_ref[...]
    o_ref[...] = jax.nn.sigmoid(z)


def _row_spec(cols):
    return pl.BlockSpec((ROWS, cols), lambda i: (i, 0))


def _full(shape):
    return pl.BlockSpec(shape, lambda i: (0,) * len(shape))


_deg_spec = pl.BlockSpec((ROWS, NW), lambda i: (i, 0))


def _s_spec(which):
    return pl.BlockSpec((1, ROWS, D), lambda i, w=which: (w, i, 0))


def kernel(x, edge_index, W1, b1, W2, b2, Wfc, bfc):
    srcm = edge_index[0].astype(jnp.int32).reshape(NW, EPT)
    dstm = edge_index[1].astype(jnp.int32).reshape(NW, EPT)
    pad = jnp.zeros((NW, EPT_PAD - EPT), jnp.int32)
    src = jnp.concatenate([srcm, pad], 1).reshape(NW, NCHUNK, CHUNK)
    dst = jnp.concatenate([dstm, pad + DPAD], 1).reshape(NW, NCHUNK, CHUNK)

    degtab = _sc_degree(edge_index[1].astype(jnp.int32))
    deg = degtab.T

    g1 = pl.pallas_call(
        _tc_in,
        grid=(N // ROWS,),
        in_specs=[_row_spec(D), _full((D, D)), _deg_spec],
        out_specs=_row_spec(D),
        out_shape=jax.ShapeDtypeStruct((N, D), jnp.float32),
    )(x, W1, deg)

    S1 = _sc_scatter(g1, src, dst)[:, PAD0:PAD0 + N, :]

    g2 = pl.pallas_call(
        _tc_mid,
        grid=(N // ROWS,),
        in_specs=[_s_spec(0), _s_spec(1), _row_spec(D), _deg_spec,
                  _full((D, D)), _full((1, D))],
        out_specs=_row_spec(D),
        out_shape=jax.ShapeDtypeStruct((N, D), jnp.float32),
    )(S1, S1, g1, deg, W2, b1.reshape(1, D))

    S2 = _sc_scatter(g2, src, dst)[:, PAD0:PAD0 + N, :]

    out = pl.pallas_call(
        _tc_out,
        grid=(N // ROWS,),
        in_specs=[_s_spec(0), _s_spec(1), _row_spec(D), _deg_spec,
                  _full((D, N_COMM)), _full((1, D)), _full((1, N_COMM))],
        out_specs=_row_spec(N_COMM),
        out_shape=jax.ShapeDtypeStruct((N, N_COMM), jnp.float32),
    )(S2, S2, g2, deg, Wfc, b2.reshape(1, D), bfc.reshape(1, N_COMM))

    return out


# speedup vs baseline: 1.6409x; 1.6409x over previous
import functools

import jax
import jax.numpy as jnp
from jax import lax
from jax.experimental import pallas as pl
from jax.experimental.pallas import tpu as pltpu
from jax.experimental.pallas import tpu_sc as plsc

N = 10000
E = 320000
D = 128
N_COMM = 64

NC = 2
NS = 16
NW = NC * NS
L = 16
EPT = E // NW
CHUNK = 80
NCHUNK = EPT // CHUNK
N_PAD = 10240
PAD0 = 120
HALF = N_PAD // 2
HRPT = HALF // NS
RPT = N_PAD // NS
ZCH = 160

_mesh = plsc.VectorSubcoreMesh(core_axis_name="c", subcore_axis_name="s")


@functools.partial(
    pl.kernel,
    mesh=_mesh,
    out_type=jax.ShapeDtypeStruct((NW, N), jnp.float32),
    compiler_params=pltpu.CompilerParams(needs_layout_passes=False),
    scratch_types=[
        pltpu.VMEM((EPT,), jnp.int32),
        pltpu.VMEM((N,), jnp.float32),
    ],
)
def _sc_degree(dst_hbm, out_hbm, idx_v, hist):
    c = lax.axis_index("c")
    s = lax.axis_index("s")
    wid = c * NS + s

    def _fill(i, _):
        hist[pl.ds(i * L, L)] = jnp.zeros((L,), jnp.float32)
        return 0

    lax.fori_loop(0, N // L, _fill, 0)
    pltpu.sync_copy(dst_hbm.at[pl.ds(wid * EPT, EPT)], idx_v)
    ones = jnp.ones((L,), jnp.float32)

    def _step(j, _):
        d = idx_v[pl.ds(j * L, L)]
        plsc.addupdate_scatter(hist, [d], ones)
        return 0

    lax.fori_loop(0, EPT // L, _step, 0)
    pltpu.sync_copy(hist, out_hbm.at[wid])


@functools.partial(
    pl.kernel,
    mesh=_mesh,
    out_type=jax.ShapeDtypeStruct((NC, N_PAD, D), jnp.float32),
    scratch_types=[
        pltpu.VMEM((NCHUNK, CHUNK), jnp.int32),
        pltpu.VMEM((NCHUNK, CHUNK), jnp.int32),
        pltpu.VMEM((NCHUNK, CHUNK), jnp.int32),
        pltpu.VMEM((CHUNK, D), jnp.float32),
        pltpu.VMEM((CHUNK, D), jnp.float32),
        pltpu.VMEM((ZCH, D), jnp.float32),
        pltpu.VMEM_SHARED((HALF, D), jnp.float32),
        pltpu.SemaphoreType.DMA,
        pltpu.SemaphoreType.DMA,
    ],
)
def _sc_scatter(g_hbm, src_hbm, dst_hbm, out_hbm,
                src_v, dst_v, loc_v, rows_a, rows_b, zero_v, acc, sem_a,
                sem_b):
    c = lax.axis_index("c")
    s = lax.axis_index("s")
    wid = c * NS + s

    def _fill(i, _):
        for k in range(D // L):
            zero_v[i, pl.ds(k * L, L)] = jnp.zeros((L,), jnp.float32)
        return 0

    lax.fori_loop(0, ZCH, _fill, 0)
    pltpu.sync_copy(src_hbm.at[wid], src_v)
    pltpu.sync_copy(dst_hbm.at[wid], dst_v)

    for p in range(2):
        base = p * HALF

        trash = (HALF - 1) * p

        def _localize(j, _):
            for q in range(CHUNK // L):
                r = dst_v[j, pl.ds(q * L, L)] + (PAD0 - base)
                ok = (r >= 0) & (r < HALF)
                loc_v[j, pl.ds(q * L, L)] = jnp.where(ok, r, trash)
            return 0

        lax.fori_loop(0, NCHUNK, _localize, 0)

        for t in range(HRPT // ZCH):
            pltpu.sync_copy(zero_v, acc.at[pl.ds(s * HRPT + t * ZCH, ZCH)])
        plsc.subcore_barrier()

        pltpu.async_copy(g_hbm.at[src_v.at[0]], rows_a, sem_a)

        def _step(i, _):
            j = 2 * i
            pltpu.make_async_copy(g_hbm.at[src_v.at[j]], rows_a, sem_a).wait()
            pltpu.async_copy(g_hbm.at[src_v.at[j + 1]], rows_b, sem_b)
            pltpu.sync_copy(rows_a, acc.at[loc_v.at[j]], add=True)
            pltpu.make_async_copy(
                g_hbm.at[src_v.at[j + 1]], rows_b, sem_b).wait()
            pltpu.async_copy(g_hbm.at[src_v.at[j + 2]], rows_a, sem_a)
            pltpu.sync_copy(rows_b, acc.at[loc_v.at[j + 1]], add=True)
            return 0

        lax.fori_loop(0, (NCHUNK - 1) // 2, _step, 0)
        pltpu.make_async_copy(
            g_hbm.at[src_v.at[NCHUNK - 1]], rows_a, sem_a).wait()
        pltpu.sync_copy(rows_a, acc.at[loc_v.at[NCHUNK - 1]], add=True)
        plsc.subcore_barrier()
        pltpu.sync_copy(acc.at[pl.ds(s * HRPT, HRPT)],
                        out_hbm.at[c, pl.ds(base + s * HRPT, HRPT)])
        plsc.subcore_barrier()


ROWS = 1000


def _dinv(deg_ref):
    deg = jnp.sum(deg_ref[...], axis=1) + 1.0
    return lax.rsqrt(deg)


def _tc_in(x_ref, w_ref, deg_ref, o_ref):
    di = _dinv(deg_ref)
    o_ref[...] = jnp.dot(x_ref[...], w_ref[...],
                         preferred_element_type=jnp.float32) * di[:, None]


def _tc_mid(s0_ref, s1_ref, g_ref, deg_ref, w_ref, b_ref, o_ref):
    di = _dinv(deg_ref)
    h = (s0_ref[0] + s1_ref[0] + g_ref[...]) * di[:, None] + b_ref[...]
    h = jnp.maximum(h, 0.0)
    o_ref[...] = jnp.dot(h, w_ref[...],
                         preferred_element_type=jnp.float32) * di[:, None]


def _tc_out(s0_ref, s1_ref, g_ref, deg_ref, w_ref, b_ref, bfc_ref, o_ref):
    di = _dinv(deg_ref)
    h = (s0_ref[0] + s1_ref[0] + g_ref[...]) * di[:, None] + b_ref[...]
    h = jnp.maximum(h, 0.0)
    z = jnp.dot(h, w_ref[...], preferred_element_type=jnp.float32) + bfc_ref[...]
    o_ref[...] = jax.nn.sigmoid(z)


def _row_spec(cols):
    return pl.BlockSpec((ROWS, cols), lambda i: (i, 0))


def _full(shape):
    return pl.BlockSpec(shape, lambda i: (0,) * len(shape))


_deg_spec = pl.BlockSpec((ROWS, NW), lambda i: (i, 0))


def _s_spec(which):
    return pl.BlockSpec((1, ROWS, D), lambda i, w=which: (w, i, 0))


def kernel(x, edge_index, W1, b1, W2, b2, Wfc, bfc):
    src = edge_index[0].astype(jnp.int32).reshape(NW, NCHUNK, CHUNK)
    dst = edge_index[1].astype(jnp.int32).reshape(NW, NCHUNK, CHUNK)

    degtab = _sc_degree(edge_index[1].astype(jnp.int32))
    deg = degtab.T

    g1 = pl.pallas_call(
        _tc_in,
        grid=(N // ROWS,),
        in_specs=[_row_spec(D), _full((D, D)), _deg_spec],
        out_specs=_row_spec(D),
        out_shape=jax.ShapeDtypeStruct((N, D), jnp.float32),
    )(x, W1, deg)

    S1 = _sc_scatter(g1, src, dst)[:, PAD0:PAD0 + N, :]

    g2 = pl.pallas_call(
        _tc_mid,
        grid=(N // ROWS,),
        in_specs=[_s_spec(0), _s_spec(1), _row_spec(D), _deg_spec,
                  _full((D, D)), _full((1, D))],
        out_specs=_row_spec(D),
        out_shape=jax.ShapeDtypeStruct((N, D), jnp.float32),
    )(S1, S1, g1, deg, W2, b1.reshape(1, D))

    S2 = _sc_scatter(g2, src, dst)[:, PAD0:PAD0 + N, :]

    out = pl.pallas_call(
        _tc_out,
        grid=(N // ROWS,),
        in_specs=[_s_spec(0), _s_spec(1), _row_spec(D), _deg_spec,
                  _full((D, N_COMM)), _full((1, D)), _full((1, N_COMM))],
        out_specs=_row_spec(N_COMM),
        out_shape=jax.ShapeDtypeStruct((N, N_COMM), jnp.float32),
    )(S2, S2, g2, deg, Wfc, b2.reshape(1, D), bfc.reshape(1, N_COMM))

    return out
